# baseline (speedup 1.0000x reference)
import functools

import jax
import jax.numpy as jnp
from jax import lax
from jax.experimental import pallas as pl
from jax.experimental.pallas import tpu as pltpu
from jax.experimental.pallas import tpu_sc as plsc

NUM_CORES = 2
NUM_SUBCORES = 16
NUM_WORKERS = NUM_CORES * NUM_SUBCORES
LANES = 16

CB = 256
W_B = 512
L_BLK = 8

MAX_NORM = 1.0


def _body(
    xt_hbm,
    table_hbm,
    out_hbm,
    idx_v,
    rows_v,
    out_v,
    sem_idx,
    sem_g0,
    sem_g1,
    sem_w0,
    sem_w1,
):
    d_model = table_hbm.shape[1]
    n_l = out_hbm.shape[0]
    n_chunks = n_l * 2
    chunks_per_ss = 2 * L_BLK
    n_ss = n_chunks // chunks_per_ss
    wid = lax.axis_index("s") * NUM_CORES + lax.axis_index("c")
    b0w = wid * W_B
    sem_g = (sem_g0, sem_g1)
    sem_w = (sem_w0, sem_w1)

    def params(s):
        ss = s // chunks_per_ss
        c = s % chunks_per_ss
        li = c // 2
        h = c % 2
        return ss, c, li, h, ss * L_BLK + li

    def fire_gather(s, p):
        ss, c, li, h, l = params(s)
        par = ss % 2
        for pb in (0, 1):
            @pl.when(par == pb)
            def _():
                for k in range(CB // 128):
                    pltpu.async_copy(
                        table_hbm.at[
                            idx_v.at[pb, li, pl.ds(h * CB + k * 128, 128)]
                        ],
                        rows_v.at[p, pl.ds(k * 128, 128)],
                        sem_g[p],
                    )

    def wait_gather(p):
        for k in range(CB // 128):
            pltpu.make_async_copy(
                table_hbm.at[idx_v.at[0, 0, pl.ds(k * 128, 128)]],
                rows_v.at[p, pl.ds(k * 128, 128)],
                sem_g[p],
            ).wait()

    def fire_idx(ss_next):
        par = ss_next % 2
        for pb in (0, 1):
            @pl.when(par == pb)
            def _():
                pltpu.async_copy(
                    xt_hbm.at[
                        pl.ds(ss_next * L_BLK, L_BLK), pl.ds(b0w, W_B)
                    ],
                    idx_v.at[pb],
                    sem_idx,
                )

    def wait_idx():
        pltpu.make_async_copy(
            xt_hbm.at[pl.ds(0, L_BLK), pl.ds(b0w, W_B)],
            idx_v.at[0],
            sem_idx,
        ).wait()

    def fire_writeback(s, p):
        ss, c, li, h, l = params(s)
        pltpu.async_copy(
            out_v.at[p],
            out_hbm.at[l, :, pl.ds(b0w + h * CB, CB)],
            sem_w[p],
        )

    def wait_writeback(p):
        pltpu.make_async_copy(
            out_v.at[p],
            out_hbm.at[0, :, pl.ds(0, CB)],
            sem_w[p],
        ).wait()

    def compute(p):
        rp = rows_v.at[p]
        ov = out_v.at[p]
        r_iota = lax.iota(jnp.int32, LANES)

        def group_fn(g, carry):
            r_vec = g * LANES + r_iota
            accs = [jnp.zeros((LANES,), jnp.float32) for _ in range(4)]
            for d in range(d_model):
                v = plsc.load_gather(
                    rp, [r_vec, jnp.full((LANES,), d, jnp.int32)]
                )
                accs[d % 4] = accs[d % 4] + jnp.abs(v)
            norm = (accs[0] + accs[1]) + (accs[2] + accs[3])
            scale = jnp.where(
                norm > MAX_NORM,
                MAX_NORM / (norm + 1e-7),
                jnp.float32(1.0),
            )
            c_vec = g * LANES + r_iota
            for d in range(d_model):
                v = plsc.load_gather(
                    rp, [r_vec, jnp.full((LANES,), d, jnp.int32)]
                )
                plsc.store_scatter(
                    ov,
                    [jnp.full((LANES,), d, jnp.int32), c_vec],
                    v * scale,
                )
            return carry

        lax.fori_loop(0, CB // LANES, group_fn, 0)

    pltpu.sync_copy(
        xt_hbm.at[pl.ds(0, L_BLK), pl.ds(b0w, W_B)], idx_v.at[0]
    )
    fire_gather(0, 0)
    fire_gather(1, 1)

    def step_fn(k, carry):
        for p in (0, 1):
            s = 2 * k + p
            ss, c, li, h, l = params(s)
            wait_gather(p)

            @pl.when(s >= 2)
            def _():
                wait_writeback(p)

            compute(p)
            fire_writeback(s, p)

            if p == 0:
                @pl.when(jnp.logical_and(c == 6, ss + 1 < n_ss))
                def _():
                    fire_idx(ss + 1)

                @pl.when(jnp.logical_and(c == 14, ss + 1 < n_ss))
                def _():
                    wait_idx()

            @pl.when(s + 2 < n_chunks)
            def _():
                fire_gather(s + 2, p)
        return carry

    lax.fori_loop(0, n_chunks // 2, step_fn, 0)
    wait_writeback(0)
    wait_writeback(1)


def kernel(x, table):
    batch, hist = x.shape
    vocab, d_model = table.shape
    xt = x.T.astype(jnp.int32)

    mesh = plsc.VectorSubcoreMesh(
        core_axis_name="c",
        subcore_axis_name="s",
        num_cores=NUM_CORES,
        num_subcores=NUM_SUBCORES,
    )
    run = functools.partial(
        pl.kernel,
        out_type=jax.ShapeDtypeStruct((hist, d_model, batch), jnp.float32),
        mesh=mesh,
        compiler_params=pltpu.CompilerParams(
            needs_layout_passes=False, use_tc_tiling_on_sc=False
        ),
        scratch_types=[
            pltpu.VMEM((2, L_BLK, W_B), jnp.int32),
            pltpu.VMEM((2, CB, d_model), jnp.float32),
            pltpu.VMEM((2, d_model, CB), jnp.float32),
            pltpu.SemaphoreType.DMA,
            pltpu.SemaphoreType.DMA,
            pltpu.SemaphoreType.DMA,
            pltpu.SemaphoreType.DMA,
            pltpu.SemaphoreType.DMA,
        ],
    )(_body)
    out_t = run(xt, table)
    return out_t.transpose(2, 0, 1)


# speedup vs baseline: 1.8180x; 1.8180x over previous
import functools

import jax
import jax.numpy as jnp
from jax import lax
from jax.experimental import pallas as pl
from jax.experimental.pallas import tpu as pltpu
from jax.experimental.pallas import tpu_sc as plsc

NUM_CORES = 2
NUM_SUBCORES = 16
NUM_WORKERS = NUM_CORES * NUM_SUBCORES

CB = 256
W_B = 512
L_BLK = 8
NBUF = 4
PAD_D = 128

MAX_NORM = 1.0
BT = 512


def _sc_body(
    xt_hbm,
    table_hbm,
    inter_hbm,
    idx_v,
    rows_v,
    sem_idx,
    sem_g0,
    sem_g1,
    sem_g2,
    sem_g3,
    sem_w0,
    sem_w1,
    sem_w2,
    sem_w3,
):
    d_model = table_hbm.shape[1]
    n_l = xt_hbm.shape[0]
    n_chunks = n_l * 2
    chunks_per_ss = 2 * L_BLK
    n_ss = n_chunks // chunks_per_ss
    wid = lax.axis_index("s") * NUM_CORES + lax.axis_index("c")
    b0w = wid * W_B
    sem_g = (sem_g0, sem_g1, sem_g2, sem_g3)
    sem_w = (sem_w0, sem_w1, sem_w2, sem_w3)

    def params(s):
        ss = s // chunks_per_ss
        c = s % chunks_per_ss
        li = c // 2
        h = c % 2
        return ss, c, li, h, ss * L_BLK + li

    def fire_gather(s, j):
        ss, c, li, h, l = params(s)
        par = ss % 2
        for pb in (0, 1):
            @pl.when(par == pb)
            def _():
                for k in range(CB // 128):
                    pltpu.async_copy(
                        table_hbm.at[
                            idx_v.at[pb, li, pl.ds(h * CB + k * 128, 128)]
                        ],
                        rows_v.at[j, pl.ds(k * 128, 128)],
                        sem_g[j],
                    )

    def wait_gather(j):
        for k in range(CB // 128):
            pltpu.make_async_copy(
                table_hbm.at[idx_v.at[0, 0, pl.ds(k * 128, 128)]],
                rows_v.at[j, pl.ds(k * 128, 128)],
                sem_g[j],
            ).wait()

    def fire_idx(ss_next):
        par = ss_next % 2
        for pb in (0, 1):
            @pl.when(par == pb)
            def _():
                pltpu.async_copy(
                    xt_hbm.at[
                        pl.ds(ss_next * L_BLK, L_BLK), pl.ds(b0w, W_B)
                    ],
                    idx_v.at[pb],
                    sem_idx,
                )

    def wait_idx():
        pltpu.make_async_copy(
            xt_hbm.at[pl.ds(0, L_BLK), pl.ds(b0w, W_B)],
            idx_v.at[0],
            sem_idx,
        ).wait()

    def fire_writeback(s, j):
        ss, c, li, h, l = params(s)
        pltpu.async_copy(
            rows_v.at[j],
            inter_hbm.at[
                l, pl.ds(b0w + h * CB, CB), pl.ds(0, d_model)
            ],
            sem_w[j],
        )

    def wait_writeback(j):
        pltpu.make_async_copy(
            rows_v.at[j],
            inter_hbm.at[0, pl.ds(0, CB), pl.ds(0, d_model)],
            sem_w[j],
        ).wait()

    pltpu.sync_copy(
        xt_hbm.at[pl.ds(0, L_BLK), pl.ds(b0w, W_B)], idx_v.at[0]
    )
    fire_gather(0, 0)
    fire_gather(1, 1)

    def step_fn(k, carry):
        for j in range(NBUF):
            s = 4 * k + j
            ss, c, li, h, l = params(s)
            wait_gather(j)
            fire_writeback(s, j)

            if j == 2:
                @pl.when(jnp.logical_and(c == 6, ss + 1 < n_ss))
                def _():
                    fire_idx(ss + 1)

                @pl.when(jnp.logical_and(c == 14, ss + 1 < n_ss))
                def _():
                    wait_idx()

            j2 = (j + 2) % NBUF

            @pl.when(s + 2 < n_chunks)
            def _():
                @pl.when(s >= 2)
                def _():
                    wait_writeback(j2)

                fire_gather(s + 2, j2)
        return carry

    lax.fori_loop(0, n_chunks // NBUF, step_fn, 0)
    for j in range(NBUF):
        wait_writeback(j)


def _tc_body(in_ref, out_ref):
    blk = in_ref[0]
    lane = lax.broadcasted_iota(jnp.int32, (1, PAD_D), 1)
    rows = jnp.where(lane < out_ref.shape[1], blk, jnp.float32(0.0))
    norm = jnp.sum(jnp.abs(rows), axis=1, keepdims=True)
    scale = jnp.where(
        norm > MAX_NORM, MAX_NORM / (norm + 1e-7), jnp.float32(1.0)
    )
    t = (rows * scale).T
    out_ref[0] = t[: out_ref.shape[1], :]


def kernel(x, table):
    batch, hist = x.shape
    vocab, d_model = table.shape
    xt = x.T.astype(jnp.int32)

    mesh = plsc.VectorSubcoreMesh(
        core_axis_name="c",
        subcore_axis_name="s",
        num_cores=NUM_CORES,
        num_subcores=NUM_SUBCORES,
    )
    gather = functools.partial(
        pl.kernel,
        out_type=jax.ShapeDtypeStruct((hist, batch, PAD_D), jnp.float32),
        mesh=mesh,
        compiler_params=pltpu.CompilerParams(
            needs_layout_passes=False, use_tc_tiling_on_sc=False
        ),
        scratch_types=[
            pltpu.VMEM((2, L_BLK, W_B), jnp.int32),
            pltpu.VMEM((NBUF, CB, d_model), jnp.float32),
            pltpu.SemaphoreType.DMA,
            pltpu.SemaphoreType.DMA,
            pltpu.SemaphoreType.DMA,
            pltpu.SemaphoreType.DMA,
            pltpu.SemaphoreType.DMA,
            pltpu.SemaphoreType.DMA,
            pltpu.SemaphoreType.DMA,
            pltpu.SemaphoreType.DMA,
            pltpu.SemaphoreType.DMA,
        ],
    )(_sc_body)
    inter = gather(xt, table)

    out_t = pl.pallas_call(
        _tc_body,
        out_shape=jax.ShapeDtypeStruct((hist, d_model, batch), jnp.float32),
        grid=(hist, batch // BT),
        in_specs=[
            pl.BlockSpec((1, BT, PAD_D), lambda l, b: (l, b, 0)),
        ],
        out_specs=pl.BlockSpec((1, d_model, BT), lambda l, b: (l, 0, b)),
        compiler_params=pltpu.CompilerParams(
            dimension_semantics=("parallel", "parallel"),
        ),
    )(inter)
    return out_t.transpose(2, 0, 1)


# speedup vs baseline: 2.7742x; 1.5260x over previous
import functools

import jax
import jax.numpy as jnp
from jax import lax
from jax.experimental import pallas as pl
from jax.experimental.pallas import tpu as pltpu
from jax.experimental.pallas import tpu_sc as plsc

NUM_CORES = 2
NUM_SUBCORES = 16
NUM_WORKERS = NUM_CORES * NUM_SUBCORES

CB = 256
W_B = 512
L_BLK = 8
NBUF = 4
PAD_D = 128

MAX_NORM = 1.0
BT = 1024


def _sc_body(
    xt_hbm,
    table_hbm,
    inter_hbm,
    idx_v,
    rows_v,
    sem_idx,
    sem_g0,
    sem_g1,
    sem_g2,
    sem_g3,
    sem_w0,
    sem_w1,
    sem_w2,
    sem_w3,
):
    d_model = table_hbm.shape[1]
    n_l = xt_hbm.shape[0]
    n_chunks = n_l * 2
    chunks_per_ss = 2 * L_BLK
    n_ss = n_chunks // chunks_per_ss
    wid = lax.axis_index("s") * NUM_CORES + lax.axis_index("c")
    b0w = wid * W_B
    sem_g = (sem_g0, sem_g1, sem_g2, sem_g3)
    sem_w = (sem_w0, sem_w1, sem_w2, sem_w3)

    def params(s):
        ss = s // chunks_per_ss
        c = s % chunks_per_ss
        li = c // 2
        h = c % 2
        return ss, c, li, h, ss * L_BLK + li

    def fire_gather(s, j):
        ss, c, li, h, l = params(s)
        par = ss % 2
        for pb in (0, 1):
            @pl.when(par == pb)
            def _():
                for k in range(CB // 128):
                    pltpu.async_copy(
                        table_hbm.at[
                            idx_v.at[pb, li, pl.ds(h * CB + k * 128, 128)]
                        ],
                        rows_v.at[j, pl.ds(k * 128, 128)],
                        sem_g[j],
                    )

    def wait_gather(j):
        for k in range(CB // 128):
            pltpu.make_async_copy(
                table_hbm.at[idx_v.at[0, 0, pl.ds(k * 128, 128)]],
                rows_v.at[j, pl.ds(k * 128, 128)],
                sem_g[j],
            ).wait()

    def fire_idx(ss_next):
        par = ss_next % 2
        for pb in (0, 1):
            @pl.when(par == pb)
            def _():
                pltpu.async_copy(
                    xt_hbm.at[
                        pl.ds(ss_next * L_BLK, L_BLK), pl.ds(b0w, W_B)
                    ],
                    idx_v.at[pb],
                    sem_idx,
                )

    def wait_idx():
        pltpu.make_async_copy(
            xt_hbm.at[pl.ds(0, L_BLK), pl.ds(b0w, W_B)],
            idx_v.at[0],
            sem_idx,
        ).wait()

    def fire_writeback(s, j):
        ss, c, li, h, l = params(s)
        pltpu.async_copy(
            rows_v.at[j],
            inter_hbm.at[
                l, pl.ds(b0w + h * CB, CB), pl.ds(0, d_model)
            ],
            sem_w[j],
        )

    def wait_writeback(j):
        pltpu.make_async_copy(
            rows_v.at[j],
            inter_hbm.at[0, pl.ds(0, CB), pl.ds(0, d_model)],
            sem_w[j],
        ).wait()

    pltpu.sync_copy(
        xt_hbm.at[pl.ds(0, L_BLK), pl.ds(b0w, W_B)], idx_v.at[0]
    )
    fire_gather(0, 0)
    fire_gather(1, 1)

    def step_fn(k, carry):
        for j in range(NBUF):
            s = 4 * k + j
            ss, c, li, h, l = params(s)
            wait_gather(j)
            fire_writeback(s, j)

            if j == 2:
                @pl.when(jnp.logical_and(c == 6, ss + 1 < n_ss))
                def _():
                    fire_idx(ss + 1)

                @pl.when(jnp.logical_and(c == 14, ss + 1 < n_ss))
                def _():
                    wait_idx()

            j2 = (j + 2) % NBUF

            @pl.when(s + 2 < n_chunks)
            def _():
                @pl.when(s >= 2)
                def _():
                    wait_writeback(j2)

                fire_gather(s + 2, j2)
        return carry

    lax.fori_loop(0, n_chunks // NBUF, step_fn, 0)
    for j in range(NBUF):
        wait_writeback(j)


def _tc_body(in_ref, out_ref):
    blk = in_ref[0]
    t = blk.T[: out_ref.shape[1], :]
    norm = jnp.sum(jnp.abs(t), axis=0, keepdims=True)
    scale = jnp.where(
        norm > MAX_NORM, MAX_NORM / (norm + 1e-7), jnp.float32(1.0)
    )
    out_ref[0] = t * scale


def kernel(x, table):
    batch, hist = x.shape
    vocab, d_model = table.shape
    xt = x.T.astype(jnp.int32)

    mesh = plsc.VectorSubcoreMesh(
        core_axis_name="c",
        subcore_axis_name="s",
        num_cores=NUM_CORES,
        num_subcores=NUM_SUBCORES,
    )
    gather = functools.partial(
        pl.kernel,
        out_type=jax.ShapeDtypeStruct((hist, batch, PAD_D), jnp.float32),
        mesh=mesh,
        compiler_params=pltpu.CompilerParams(
            needs_layout_passes=False, use_tc_tiling_on_sc=False
        ),
        scratch_types=[
            pltpu.VMEM((2, L_BLK, W_B), jnp.int32),
            pltpu.VMEM((NBUF, CB, d_model), jnp.float32),
            pltpu.SemaphoreType.DMA,
            pltpu.SemaphoreType.DMA,
            pltpu.SemaphoreType.DMA,
            pltpu.SemaphoreType.DMA,
            pltpu.SemaphoreType.DMA,
            pltpu.SemaphoreType.DMA,
            pltpu.SemaphoreType.DMA,
            pltpu.SemaphoreType.DMA,
            pltpu.SemaphoreType.DMA,
        ],
    )(_sc_body)
    inter = gather(xt, table)

    out_t = pl.pallas_call(
        _tc_body,
        out_shape=jax.ShapeDtypeStruct((hist, d_model, batch), jnp.float32),
        grid=(hist, batch // BT),
        in_specs=[
            pl.BlockSpec((1, BT, PAD_D), lambda l, b: (l, b, 0)),
        ],
        out_specs=pl.BlockSpec((1, d_model, BT), lambda l, b: (l, 0, b)),
        compiler_params=pltpu.CompilerParams(
            dimension_semantics=("parallel", "parallel"),
        ),
    )(inter)
    return out_t.transpose(2, 0, 1)
